# baseline (speedup 1.0000x reference)
import functools
import math

import jax
import jax.numpy as jnp
from jax import lax
from jax.experimental import pallas as pl
from jax.experimental.pallas import tpu as pltpu
from jax.experimental.pallas import tpu_sc as plsc

NC = 2
NS = 16
NW = NC * NS

DMODEL = 64
BBLK = 128
NBUF = 4
SCALE = math.sqrt(DMODEL)


def _make_gather(seq, batch):
    assert batch == NW * BBLK
    half = BBLK // 2
    mesh = plsc.VectorSubcoreMesh(
        core_axis_name="c", subcore_axis_name="s", num_cores=NC, num_subcores=NS
    )

    @functools.partial(
        pl.kernel,
        out_type=jax.ShapeDtypeStruct((seq, batch // 2, BBLK), jnp.float32),
        mesh=mesh,
        scratch_types=[
            pltpu.VMEM((seq, BBLK), jnp.int32),
            pltpu.VMEM((NBUF, BBLK, DMODEL), jnp.float32),
            pltpu.SemaphoreType.DMA((NBUF,)),
            pltpu.SemaphoreType.DMA((NBUF,)),
        ],
        compiler_params=pltpu.CompilerParams(
            needs_layout_passes=False,
            disable_bounds_checks=True,
            use_tc_tiling_on_sc=False,
        ),
    )
    def gather(xt_hbm, tab_hbm, out_hbm, idx_v, gbuf, gsem, wsem):
        wid = lax.axis_index("s") * NC + lax.axis_index("c")
        col = wid * BBLK
        row0 = wid * half
        pltpu.sync_copy(xt_hbm.at[:, pl.ds(col, BBLK)], idx_v)

        def start_gather(t, b):
            pltpu.async_copy(tab_hbm.at[idx_v.at[t]], gbuf.at[b], gsem.at[b])

        def wait_gather(b):
            pltpu.make_async_copy(
                tab_hbm.at[pl.ds(0, BBLK)], gbuf.at[b], gsem.at[b]
            ).wait()

        def start_write(t, b):
            pltpu.async_copy(
                gbuf.at[b, pl.ds(0, half), :],
                out_hbm.at[t, pl.ds(row0, half), pl.ds(0, DMODEL)],
                wsem.at[b],
            )
            pltpu.async_copy(
                gbuf.at[b, pl.ds(half, half), :],
                out_hbm.at[t, pl.ds(row0, half), pl.ds(DMODEL, DMODEL)],
                wsem.at[b],
            )

        def wait_write(b):
            for h in range(2):
                pltpu.make_async_copy(
                    gbuf.at[b, pl.ds(h * half, half), :],
                    out_hbm.at[0, pl.ds(row0, half), pl.ds(h * DMODEL, DMODEL)],
                    wsem.at[b],
                ).wait()

        for b in range(NBUF):
            start_gather(b, b)

        for b in range(NBUF):
            wait_gather(b)
            start_gather(NBUF + b, b)
            start_write(b, b)

        def round_body(r, carry):
            for b in range(NBUF):
                t = r * NBUF + b
                wait_gather(b)
                wait_write(b)
                start_gather(t + NBUF, b)
                start_write(t, b)
            return carry

        lax.fori_loop(1, seq // NBUF - 1, round_body, None)

        for b in range(NBUF):
            t = seq - NBUF + b
            wait_gather(b)
            wait_write(b)
            start_write(t, b)

        for b in range(NBUF):
            wait_write(b)

    return gather


def _tc_transpose(stage, seq, batch):
    def body(s_ref, o_ref):
        ii = lax.broadcasted_iota(jnp.int32, (DMODEL, DMODEL), 0)
        jj = lax.broadcasted_iota(jnp.int32, (DMODEL, DMODEL), 1)
        eye8 = jnp.where(ii == jj, SCALE, 0.0).astype(jnp.float32)
        blk = s_ref[0]
        for h in range(2):
            hblk = blk[:, h * DMODEL:(h + 1) * DMODEL]
            o_ref[0, :, h * DMODEL:(h + 1) * DMODEL] = lax.dot_general(
                hblk, eye8, (((0,), (0,)), ((), ())),
                preferred_element_type=jnp.float32,
            )

    return pl.pallas_call(
        body,
        grid=(seq, batch // BBLK),
        in_specs=[pl.BlockSpec((1, DMODEL, BBLK), lambda t, j: (t, j, 0))],
        out_specs=pl.BlockSpec((1, DMODEL, BBLK), lambda t, j: (t, 0, j)),
        out_shape=jax.ShapeDtypeStruct((seq, DMODEL, batch), jnp.float32),
    )(stage)


def kernel(x, table):
    b0, b1 = x.shape
    xt = x.T
    stage = _make_gather(b1, b0)(xt, table)
    out_p = _tc_transpose(stage, b1, b0)
    return out_p.transpose(2, 0, 1)


# speedup vs baseline: 3.0081x; 3.0081x over previous
import functools
import math

import jax
import jax.numpy as jnp
from jax import lax
from jax.experimental import pallas as pl
from jax.experimental.pallas import tpu as pltpu
from jax.experimental.pallas import tpu_sc as plsc

NC = 2
NS = 16
NW = NC * NS

DMODEL = 64
BBLK = 128
NBUF = 6
TROWS = 8
SCALE = math.sqrt(DMODEL)


def _make_gather(seq, batch):
    assert batch == NW * BBLK
    half = BBLK // 2
    mesh = plsc.VectorSubcoreMesh(
        core_axis_name="c", subcore_axis_name="s", num_cores=NC, num_subcores=NS
    )

    @functools.partial(
        pl.kernel,
        out_type=jax.ShapeDtypeStruct((seq, batch // 2, BBLK), jnp.float32),
        mesh=mesh,
        scratch_types=[
            pltpu.VMEM((seq, BBLK), jnp.int32),
            pltpu.VMEM((NBUF, BBLK, DMODEL), jnp.float32),
            pltpu.SemaphoreType.DMA((NBUF,)),
            pltpu.SemaphoreType.DMA((NBUF,)),
        ],
        compiler_params=pltpu.CompilerParams(
            needs_layout_passes=False,
            disable_bounds_checks=True,
            use_tc_tiling_on_sc=False,
        ),
    )
    def gather(xt_hbm, tab_hbm, out_hbm, idx_v, gbuf, gsem, wsem):
        wid = lax.axis_index("s") * NC + lax.axis_index("c")
        col = wid * BBLK
        row0 = wid * half
        pltpu.sync_copy(xt_hbm.at[:, pl.ds(col, BBLK)], idx_v)

        def start_gather(t, b):
            pltpu.async_copy(tab_hbm.at[idx_v.at[t]], gbuf.at[b], gsem.at[b])

        def wait_gather(b):
            pltpu.make_async_copy(
                tab_hbm.at[pl.ds(0, BBLK)], gbuf.at[b], gsem.at[b]
            ).wait()

        def start_write(t, b):
            pltpu.async_copy(
                gbuf.at[b, pl.ds(0, half), :],
                out_hbm.at[t, pl.ds(row0, half), pl.ds(0, DMODEL)],
                wsem.at[b],
            )
            pltpu.async_copy(
                gbuf.at[b, pl.ds(half, half), :],
                out_hbm.at[t, pl.ds(row0, half), pl.ds(DMODEL, DMODEL)],
                wsem.at[b],
            )

        def wait_write(b):
            for h in range(2):
                pltpu.make_async_copy(
                    gbuf.at[b, pl.ds(h * half, half), :],
                    out_hbm.at[0, pl.ds(row0, half), pl.ds(h * DMODEL, DMODEL)],
                    wsem.at[b],
                ).wait()

        ahead = NBUF - 2

        for b in range(ahead):
            start_gather(b, b)

        def step(t, carry):
            b = t % NBUF
            wait_gather(b)
            start_write(t, b)
            bb = (t + ahead) % NBUF
            g = t + ahead

            @pl.when(g < seq)
            def _():
                @pl.when(g >= NBUF)
                def _():
                    wait_write(bb)

                start_gather(g, bb)

            return carry

        lax.fori_loop(0, seq, step, None)

        for b in range(NBUF):
            wait_write(b)

    return gather


def _tc_transpose(stage, seq, batch):
    def body(s_ref, o_ref):
        for p in range(TROWS):
            blk = s_ref[p]
            for h in range(2):
                hblk = blk[:, h * DMODEL:(h + 1) * DMODEL]
                o_ref[p, :, h * DMODEL:(h + 1) * DMODEL] = (
                    hblk.T * jnp.float32(SCALE)
                )

    return pl.pallas_call(
        body,
        grid=(seq // TROWS, batch // BBLK),
        in_specs=[pl.BlockSpec((TROWS, DMODEL, BBLK), lambda t, j: (t, j, 0))],
        out_specs=pl.BlockSpec((TROWS, DMODEL, BBLK), lambda t, j: (t, 0, j)),
        out_shape=jax.ShapeDtypeStruct((seq, DMODEL, batch), jnp.float32),
    )(stage)


def kernel(x, table):
    b0, b1 = x.shape
    xt = x.T
    stage = _make_gather(b1, b0)(xt, table)
    out_p = _tc_transpose(stage, b1, b0)
    return out_p.transpose(2, 0, 1)


# speedup vs baseline: 3.4955x; 1.1620x over previous
import functools
import math

import jax
import jax.numpy as jnp
from jax import lax
from jax.experimental import pallas as pl
from jax.experimental.pallas import tpu as pltpu
from jax.experimental.pallas import tpu_sc as plsc

NC = 2
NS = 16
NW = NC * NS

DMODEL = 64
BBLK = 128
NBUF = 6
TROWS = 8
SCALE = math.sqrt(DMODEL)


def _make_gather(seq, batch):
    assert batch == NW * BBLK
    half = BBLK // 2
    mesh = plsc.VectorSubcoreMesh(
        core_axis_name="c", subcore_axis_name="s", num_cores=NC, num_subcores=NS
    )

    @functools.partial(
        pl.kernel,
        out_type=jax.ShapeDtypeStruct((seq, batch // 2, BBLK), jnp.float32),
        mesh=mesh,
        scratch_types=[
            pltpu.VMEM((seq, BBLK), jnp.int32),
            pltpu.VMEM((NBUF, BBLK, DMODEL), jnp.float32),
            pltpu.SemaphoreType.DMA((NBUF,)),
            pltpu.SemaphoreType.DMA((NBUF,)),
        ],
        compiler_params=pltpu.CompilerParams(
            needs_layout_passes=False,
            disable_bounds_checks=True,
            use_tc_tiling_on_sc=False,
        ),
    )
    def gather(xt_hbm, tab_hbm, out_hbm, idx_v, gbuf, gsem, wsem):
        wid = lax.axis_index("s") * NC + lax.axis_index("c")
        col = wid * BBLK
        row0 = wid * half
        pltpu.sync_copy(xt_hbm.at[:, pl.ds(col, BBLK)], idx_v)

        def start_gather(t, b):
            pltpu.async_copy(tab_hbm.at[idx_v.at[t]], gbuf.at[b], gsem.at[b])

        def wait_gather(b):
            pltpu.make_async_copy(
                tab_hbm.at[pl.ds(0, BBLK)], gbuf.at[b], gsem.at[b]
            ).wait()

        def start_write(t, b):
            pltpu.async_copy(
                gbuf.at[b, pl.ds(0, half), :],
                out_hbm.at[t, pl.ds(row0, half), pl.ds(0, DMODEL)],
                wsem.at[b],
            )
            pltpu.async_copy(
                gbuf.at[b, pl.ds(half, half), :],
                out_hbm.at[t, pl.ds(row0, half), pl.ds(DMODEL, DMODEL)],
                wsem.at[b],
            )

        def wait_write(b):
            for h in range(2):
                pltpu.make_async_copy(
                    gbuf.at[b, pl.ds(h * half, half), :],
                    out_hbm.at[0, pl.ds(row0, half), pl.ds(h * DMODEL, DMODEL)],
                    wsem.at[b],
                ).wait()

        ahead = NBUF - 2

        for b in range(ahead):
            start_gather(b, b)

        def step(t, carry):
            b = t % NBUF
            wait_gather(b)
            start_write(t, b)
            bb = (t + ahead) % NBUF
            g = t + ahead

            @pl.when(g < seq)
            def _():
                @pl.when(g >= NBUF)
                def _():
                    wait_write(bb)

                start_gather(g, bb)

            return carry

        lax.fori_loop(0, seq, step, None)

        for b in range(NBUF):
            wait_write(b)

    return gather


def _tc_transpose(stage, seq, batch):
    def body(s_ref, o_ref):
        for q in range(TROWS // 2):
            pair = s_ref[2 * q:2 * q + 2].reshape(2 * DMODEL, BBLK)
            tp = pair.T * jnp.float32(SCALE)
            for p in range(2):
                for h in range(2):
                    o_ref[2 * q + p, :, h * DMODEL:(h + 1) * DMODEL] = tp[
                        h * DMODEL:(h + 1) * DMODEL, p * DMODEL:(p + 1) * DMODEL
                    ]

    return pl.pallas_call(
        body,
        grid=(seq // TROWS, batch // BBLK),
        in_specs=[pl.BlockSpec((TROWS, DMODEL, BBLK), lambda t, j: (t, j, 0))],
        out_specs=pl.BlockSpec((TROWS, DMODEL, BBLK), lambda t, j: (t, 0, j)),
        out_shape=jax.ShapeDtypeStruct((seq, DMODEL, batch), jnp.float32),
    )(stage)


def kernel(x, table):
    b0, b1 = x.shape
    xt = x.T
    stage = _make_gather(b1, b0)(xt, table)
    out_p = _tc_transpose(stage, b1, b0)
    return out_p.transpose(2, 0, 1)


# speedup vs baseline: 4.1960x; 1.2004x over previous
import functools
import math

import jax
import jax.numpy as jnp
from jax import lax
from jax.experimental import pallas as pl
from jax.experimental.pallas import tpu as pltpu
from jax.experimental.pallas import tpu_sc as plsc

NC = 2
NS = 16
NW = NC * NS

DMODEL = 64
BBLK = 128
NBUF = 6
TROWS = 16
SCALE = math.sqrt(DMODEL)


def _make_gather(seq, batch):
    assert batch == NW * BBLK
    half = BBLK // 2
    mesh = plsc.VectorSubcoreMesh(
        core_axis_name="c", subcore_axis_name="s", num_cores=NC, num_subcores=NS
    )

    @functools.partial(
        pl.kernel,
        out_type=jax.ShapeDtypeStruct((seq, batch // 2, BBLK), jnp.float32),
        mesh=mesh,
        scratch_types=[
            pltpu.VMEM((seq, BBLK), jnp.int32),
            pltpu.VMEM((NBUF, BBLK, 2 * DMODEL), jnp.float32),
            pltpu.SemaphoreType.DMA((NBUF,)),
            pltpu.SemaphoreType.DMA((NBUF,)),
        ],
        compiler_params=pltpu.CompilerParams(
            needs_layout_passes=False,
            disable_bounds_checks=True,
            use_tc_tiling_on_sc=False,
        ),
    )
    def gather(xt_hbm, tab_hbm, out_hbm, idx_v, gbuf, gsem, wsem):
        wid = lax.axis_index("s") * NC + lax.axis_index("c")
        col = wid * BBLK
        row0 = wid * half
        pltpu.sync_copy(xt_hbm.at[:, pl.ds(col, BBLK)], idx_v)

        def start_gather(t, b):
            pltpu.async_copy(tab_hbm.at[idx_v.at[t]], gbuf.at[b], gsem.at[b])

        def wait_gather(b):
            pltpu.make_async_copy(
                tab_hbm.at[pl.ds(0, BBLK)], gbuf.at[b], gsem.at[b]
            ).wait()

        def start_write(t, b):
            for h in range(2):
                pltpu.async_copy(
                    gbuf.at[b, pl.ds(h * half, half), pl.ds(0, DMODEL)],
                    out_hbm.at[t, pl.ds(row0, half), pl.ds(h * DMODEL, DMODEL)],
                    wsem.at[b],
                )

        def wait_write(b):
            for h in range(2):
                pltpu.make_async_copy(
                    gbuf.at[b, pl.ds(h * half, half), pl.ds(0, DMODEL)],
                    out_hbm.at[0, pl.ds(row0, half), pl.ds(h * DMODEL, DMODEL)],
                    wsem.at[b],
                ).wait()

        ahead = NBUF - 2

        for b in range(ahead):
            start_gather(b, b)

        def step(t, carry):
            b = t % NBUF
            wait_gather(b)
            start_write(t, b)
            bb = (t + ahead) % NBUF
            g = t + ahead

            @pl.when(g < seq)
            def _():
                @pl.when(g >= NBUF)
                def _():
                    wait_write(bb)

                start_gather(g, bb)

            return carry

        lax.fori_loop(0, seq, step, None)

        for b in range(NBUF):
            wait_write(b)

    return gather


def _tc_transpose(stage, seq, batch):
    def body(s_ref, o_ref):
        for q in range(TROWS // 2):
            pair = s_ref[2 * q:2 * q + 2].reshape(2 * DMODEL, BBLK)
            tp = pair.T * jnp.float32(SCALE)
            for p in range(2):
                for h in range(2):
                    o_ref[2 * q + p, :, h * DMODEL:(h + 1) * DMODEL] = tp[
                        h * DMODEL:(h + 1) * DMODEL, p * DMODEL:(p + 1) * DMODEL
                    ]

    return pl.pallas_call(
        body,
        grid=(seq // TROWS, batch // BBLK),
        in_specs=[pl.BlockSpec((TROWS, DMODEL, BBLK), lambda t, j: (t, j, 0))],
        out_specs=pl.BlockSpec((TROWS, DMODEL, BBLK), lambda t, j: (t, 0, j)),
        out_shape=jax.ShapeDtypeStruct((seq, DMODEL, batch), jnp.float32),
    )(stage)


def kernel(x, table):
    b0, b1 = x.shape
    vocab, dm = table.shape
    xt = x.T
    tpad = jnp.concatenate([table, jnp.zeros((vocab, dm), jnp.float32)], axis=1)
    stage = _make_gather(b1, b0)(xt, tpad)
    out_p = _tc_transpose(stage, b1, b0)
    return out_p.transpose(2, 0, 1)
